# Initial kernel scaffold; baseline (speedup 1.0000x reference)
import jax
import jax.numpy as jnp
from jax.experimental import pallas as pl


def kernel(x_game, x_state, edge_attr, params, edge_index_gg, edge_index_ss, edge_index_hist, edge_index_in):
    raise NotImplementedError("write your pallas kernel here")


# speedup vs baseline: 1.0642x; 1.0642x over previous
import functools

import jax
import jax.numpy as jnp
from jax import lax
from jax.experimental import pallas as pl

NG = 50000
NS = 50000
E = 800000
HID = 64

_SM_ROWS = 392


def _softmax_body(x_ref, o_ref):
    x = x_ref[...]
    ridx = lax.broadcasted_iota(jnp.int32, x.shape, 0)
    cidx = lax.broadcasted_iota(jnp.int32, x.shape, 1)
    valid = (ridx * 128 + cidx) < NS
    xm = jnp.where(valid, x, -jnp.inf)
    m = jnp.max(xm)
    e = jnp.where(valid, jnp.exp(x - m), 0.0)
    o_ref[...] = e / jnp.sum(e)


def _softmax_over_all(logit):
    x = jnp.zeros((_SM_ROWS * 128,), jnp.float32).at[:NS].set(logit)
    x = x.reshape(_SM_ROWS, 128)
    y = pl.pallas_call(
        _softmax_body,
        out_shape=jax.ShapeDtypeStruct((_SM_ROWS, 128), jnp.float32),
    )(x)
    return y.reshape(-1)[:NS]


def _seg_sum(data, idx, n):
    return jax.ops.segment_sum(data, idx, num_segments=n)


def kernel(x_game, x_state, edge_attr, params, edge_index_gg, edge_index_ss, edge_index_hist, edge_index_in):
    p = params
    src, dst = edge_index_gg[0], edge_index_gg[1]
    deg = _seg_sum(jnp.ones((E,), jnp.float32), dst, NG)
    dis = jnp.where(deg > 0, 1.0 / jnp.sqrt(jnp.maximum(deg, 1.0)), 0.0)
    norm = dis[src] * dis[dst]
    out = x_game @ p['tag_W'][0]
    h = x_game
    for k in range(1, 3):
        h = _seg_sum(norm[:, None] * h[src], dst, NG)
        out = out + h @ p['tag_W'][k]
    g = jax.nn.relu(out + p['tag_b'])

    src, dst = edge_index_ss[0], edge_index_ss[1]
    s_sum = _seg_sum(x_state[src], dst, NS)
    cnt = _seg_sum(jnp.ones((E,), jnp.float32), dst, NS)
    mean = s_sum / jnp.maximum(cnt, 1.0)[:, None]
    s = jax.nn.relu(mean @ p['s2_Wl'] + x_state @ p['s2_Wr'] + p['s2_b'])

    src, dst = edge_index_hist[0], edge_index_hist[1]
    hs = g @ p['g_Ws']
    asn = hs @ p['g_as']
    adn = s @ (p['g_Wd'] @ p['g_ad'])
    aen = edge_attr @ (p['g_We'] @ p['g_ae'])
    bnd = jnp.max(asn) + jnp.max(adn) + jnp.max(aen)
    bnd = jnp.maximum(bnd, 0.2 * bnd)
    alpha = asn[src] + adn[dst] + aen
    alpha = jax.nn.leaky_relu(alpha, 0.2)
    ex = jnp.exp(alpha - bnd)
    den = _seg_sum(ex, dst, NS)
    u = _seg_sum(ex[:, None] * hs[src], dst, NS)
    s = jax.nn.relu(u / jnp.maximum(den, 1e-16)[:, None] + p['g_b'])

    src, dst = edge_index_in[0], edge_index_in[1]
    in_sum = _seg_sum(g[src], dst, NS)
    in_cnt = _seg_sum(jnp.ones((E,), jnp.float32), dst, NS)
    mean = in_sum / jnp.maximum(in_cnt, 1.0)[:, None]
    s = jax.nn.relu(mean @ p['s4_Wl'] + s @ p['s4_Wr'] + p['s4_b'])

    s = s @ p['lin_W'] + p['lin_b']
    logit = (s @ p['ll_W'] + p['ll_b'])[:, 0]
    state_out = _softmax_over_all(logit)[:, None]
    return (state_out, x_game)


# speedup vs baseline: 5.9434x; 5.5849x over previous
import functools

import jax
import jax.numpy as jnp
from jax import lax
from jax.experimental import pallas as pl
from jax.experimental.pallas import tpu as pltpu
from jax.experimental.pallas import tpu_sc as plsc

NG = 50000
NS = 50000
E = 800000
HID = 64

NSUB = 16
NCH = E // 128
CH_PER_TILE = NCH // NSUB
CH_EXTRA = NCH % NSUB
HALF = NS // 2
ACC_ROWS = 25088
ZROWS_PER_TILE = ACC_ROWS // NSUB
TRASH = 25024
OUT_PER_TILE = 1568
ZCH = 224

_MESH = plsc.VectorSubcoreMesh(core_axis_name="c", subcore_axis_name="s")


def _nchunks(s_idx):
    return jnp.where(s_idx < CH_EXTRA, CH_PER_TILE + 1, CH_PER_TILE)


def _compute_local_dst(dstb, ldb, base):
    def j_body(j, _):
        off = pl.multiple_of(j * 16, 16)
        d16 = dstb[pl.ds(off, 16)]
        ld = d16 - base
        ok = (ld >= 0) & (ld < HALF)
        ldb[pl.ds(off, 16)] = jnp.where(ok, ld, TRASH)
        return 0
    lax.fori_loop(0, 8, j_body, 0)


def _zero_acc(zsrc, acc, s_idx):
    z0 = s_idx * ZROWS_PER_TILE
    for i in range(ZROWS_PER_TILE // ZCH):
        pltpu.sync_copy(zsrc, acc.at[pl.ds(z0 + i * ZCH, ZCH)])


def _write_out(acc, out, c_idx, s_idx):
    @pl.when(s_idx < NSUB - 1)
    def _():
        r0 = pl.multiple_of(s_idx * OUT_PER_TILE, 8)
        o0 = pl.multiple_of(c_idx * HALF + s_idx * OUT_PER_TILE, 8)
        pltpu.sync_copy(acc.at[pl.ds(r0, OUT_PER_TILE)],
                        out.at[pl.ds(o0, OUT_PER_TILE)])

    last0 = (NSUB - 1) * OUT_PER_TILE
    @pl.when(s_idx == NSUB - 1)
    def _():
        o0 = pl.multiple_of(c_idx * HALF + last0, 8)
        pltpu.sync_copy(acc.at[pl.ds(last0, HALF - last0)],
                        out.at[pl.ds(o0, HALF - last0)])


def _scale_rows(rows, wref, w, densb=None):
    ngrp = w // 16

    def g_body(jg, _):
        off = pl.multiple_of(jg * 16, 16)
        w16 = wref[pl.ds(off, 16)]
        for r in range(16):
            sp = jnp.full((16,), w16[r], jnp.float32)
            row = jg * 16 + r
            if densb is not None:
                densb[row, :] = sp
            for j in range(ngrp):
                o2 = pl.multiple_of(j * 16, 16)
                rows[row, pl.ds(o2, 16)] = rows[row, pl.ds(o2, 16)] * sp
        return 0
    lax.fori_loop(0, 8, g_body, 0)


def _make_seg_kernel(mode, w):
    with_rows = mode in ("plain", "norm", "gat")
    with_cnt = mode in ("plain", "count")
    cw = 8 if with_cnt else 16

    out_type = []
    if with_rows:
        out_type.append(jax.ShapeDtypeStruct((NS, w), jnp.float32))
    if with_cnt or mode == "wcount":
        out_type.append(jax.ShapeDtypeStruct((NS, cw), jnp.float32))

    scratch = [pltpu.VMEM((128,), jnp.int32)]
    scratch.append(pltpu.VMEM((128,), jnp.int32))
    if with_rows or mode == "wcount":
        scratch.append(pltpu.VMEM((128,), jnp.int32))
    if with_rows:
        scratch.append(pltpu.VMEM((128, w), jnp.float32))
        scratch.append(pltpu.VMEM_SHARED((ACC_ROWS, w), jnp.float32))
    if with_cnt:
        scratch.append(pltpu.VMEM_SHARED((ACC_ROWS, 8), jnp.float32))
        scratch.append(pltpu.VMEM((128, 8), jnp.float32))
    if mode == "wcount":
        scratch.append(pltpu.VMEM_SHARED((ACC_ROWS, 16), jnp.float32))
        scratch.append(pltpu.VMEM((128, 16), jnp.float32))
    if mode == "norm":
        scratch.append(pltpu.VMEM((128,), jnp.float32))
        scratch.append(pltpu.VMEM((128,), jnp.float32))
        scratch.append(pltpu.VMEM((128,), jnp.float32))
    if mode in ("gat", "wcount"):
        scratch.append(pltpu.VMEM((128,), jnp.float32))
        scratch.append(pltpu.VMEM((128,), jnp.float32))
        scratch.append(pltpu.VMEM((128,), jnp.float32))
        scratch.append(pltpu.VMEM((128,), jnp.float32))
        scratch.append(pltpu.VMEM((16,), jnp.float32))
    scratch.append(pltpu.SemaphoreType.DMA)
    scratch.append(pltpu.SemaphoreType.DMA)
    scratch.append(pltpu.SemaphoreType.DMA)

    def body(*refs):
        it = iter(refs)
        if with_rows:
            table = next(it)
        if mode == "norm":
            wnode = next(it)
        if mode in ("gat", "wcount"):
            asn, adn, aef, bnd = next(it), next(it), next(it), next(it)
        srcf = next(it) if (with_rows or mode == "wcount") else None
        dstf = next(it)
        zrows = next(it) if with_rows else None
        zcnt = next(it) if (with_cnt or mode == "wcount") else None
        ones8 = next(it) if with_cnt else None
        out_rows = next(it) if with_rows else None
        out_cnt = next(it) if (with_cnt or mode == "wcount") else None
        dstb = next(it)
        ldb = next(it)
        srcb = next(it) if (with_rows or mode == "wcount") else None
        if with_rows:
            rows = next(it)
            acc = next(it)
        if with_cnt:
            cacc = next(it)
            onesb = next(it)
        if mode == "wcount":
            cacc = next(it)
            densb = next(it)
        if mode == "norm":
            wsv, wdv, wb = next(it), next(it), next(it)
        if mode in ("gat", "wcount"):
            asv, adv, aev, exb, bndv = (next(it), next(it), next(it),
                                        next(it), next(it))
        sem = next(it)
        sem2 = next(it)
        sem3 = next(it)

        c_idx = lax.axis_index("c")
        s_idx = lax.axis_index("s")
        base = c_idx * HALF

        if with_rows:
            _zero_acc(zrows, acc, s_idx)
        if with_cnt or mode == "wcount":
            _zero_acc(zcnt, cacc, s_idx)
        if with_cnt:
            pltpu.sync_copy(ones8, onesb)
        if mode in ("gat", "wcount"):
            pltpu.sync_copy(bnd, bndv)
        plsc.subcore_barrier()

        def alpha_chunk():
            bb = bndv[...]
            def jg_body(j, _):
                off = pl.multiple_of(j * 16, 16)
                t = asv[pl.ds(off, 16)] + adv[pl.ds(off, 16)] + aev[pl.ds(off, 16)]
                t = jnp.maximum(t, 0.2 * t)
                exb[pl.ds(off, 16)] = jnp.exp(t - bb)
                return 0
            lax.fori_loop(0, 8, jg_body, 0)

        def chunk_body(k, _):
            chunk = s_idx + NSUB * k
            e0 = pl.multiple_of(chunk * 128, 128)
            pltpu.sync_copy(dstf.at[pl.ds(e0, 128)], dstb)
            if srcb is not None:
                pltpu.sync_copy(srcf.at[pl.ds(e0, 128)], srcb)
            if mode in ("gat", "wcount"):
                pltpu.sync_copy(aef.at[pl.ds(e0, 128)], aev)
            _compute_local_dst(dstb, ldb, base)

            if with_rows:
                h = pltpu.async_copy(table.at[srcb], rows, sem)
            if mode == "norm":
                h2 = pltpu.async_copy(wnode.at[srcb], wsv, sem2)
                h3 = pltpu.async_copy(wnode.at[dstb], wdv, sem3)
                h2.wait(); h3.wait()
            if mode in ("gat", "wcount"):
                h2 = pltpu.async_copy(asn.at[srcb], asv, sem2)
                h3 = pltpu.async_copy(adn.at[dstb], adv, sem3)
                h2.wait(); h3.wait()
            if with_rows:
                h.wait()

            if mode == "norm":
                def jn_body(j, _):
                    off = pl.multiple_of(j * 16, 16)
                    wb[pl.ds(off, 16)] = wsv[pl.ds(off, 16)] * wdv[pl.ds(off, 16)]
                    return 0
                lax.fori_loop(0, 8, jn_body, 0)
                _scale_rows(rows, wb, w)
            if mode == "gat":
                alpha_chunk()
                _scale_rows(rows, exb, w)
            if mode == "wcount":
                alpha_chunk()
                def js_body(jg, _):
                    off = pl.multiple_of(jg * 16, 16)
                    w16 = exb[pl.ds(off, 16)]
                    for r in range(16):
                        densb[jg * 16 + r, :] = jnp.full((16,), w16[r], jnp.float32)
                    return 0
                lax.fori_loop(0, 8, js_body, 0)

            if with_rows:
                pltpu.sync_copy(rows, acc.at[ldb], add=True)
            if with_cnt:
                pltpu.sync_copy(onesb, cacc.at[ldb], add=True)
            if mode == "wcount":
                pltpu.sync_copy(densb, cacc.at[ldb], add=True)
            return 0

        lax.fori_loop(0, _nchunks(s_idx), chunk_body, 0)
        plsc.subcore_barrier()
        if with_rows:
            _write_out(acc, out_rows, c_idx, s_idx)
        if with_cnt or mode == "wcount":
            _write_out(cacc, out_cnt, c_idx, s_idx)

    if len(out_type) == 1:
        out_type = out_type[0]
    return functools.partial(
        pl.kernel, mesh=_MESH, out_type=out_type, scratch_types=scratch,
        compiler_params=pltpu.CompilerParams(use_tc_tiling_on_sc=False),
    )(body)


_seg_plain_64 = _make_seg_kernel("plain", 64)
_seg_plain_16 = _make_seg_kernel("plain", 16)
_seg_norm_16 = _make_seg_kernel("norm", 16)
_seg_gat = _make_seg_kernel("gat", 64)
_seg_count = _make_seg_kernel("count", 0)
_seg_wcount = _make_seg_kernel("wcount", 0)

_SM_ROWS = 392


def _softmax_body(x_ref, o_ref):
    x = x_ref[...]
    ridx = lax.broadcasted_iota(jnp.int32, x.shape, 0)
    cidx = lax.broadcasted_iota(jnp.int32, x.shape, 1)
    valid = (ridx * 128 + cidx) < NS
    xm = jnp.where(valid, x, -jnp.inf)
    m = jnp.max(xm)
    e = jnp.where(valid, jnp.exp(x - m), 0.0)
    o_ref[...] = e / jnp.sum(e)


def _softmax_over_all(logit):
    x = jnp.zeros((_SM_ROWS * 128,), jnp.float32).at[:NS].set(logit)
    x = x.reshape(_SM_ROWS, 128)
    y = pl.pallas_call(
        _softmax_body,
        out_shape=jax.ShapeDtypeStruct((_SM_ROWS, 128), jnp.float32),
    )(x)
    return y.reshape(-1)[:NS]


def _pad16(x):
    return jnp.pad(x, ((0, 0), (0, 16 - x.shape[1])))


def kernel(x_game, x_state, edge_attr, params, edge_index_gg, edge_index_ss, edge_index_hist, edge_index_in):
    p = params
    e2 = lambda v: v
    _Z64 = jnp.zeros((ZCH, 64), jnp.float32)
    _Z16 = jnp.zeros((ZCH, 16), jnp.float32)
    _Z8 = jnp.zeros((ZCH, 8), jnp.float32)
    _ONES8 = jnp.ones((128, 8), jnp.float32)
    fold = lambda c: c[:, 0]

    deg = fold(_seg_count(e2(edge_index_gg[1]), _Z8, _ONES8))
    dis = jnp.where(deg > 0, 1.0 / jnp.sqrt(jnp.maximum(deg, 1.0)), 0.0)
    x16 = _pad16(x_game)
    h1 = _seg_norm_16(x16, dis, e2(edge_index_gg[0]), e2(edge_index_gg[1]), _Z16)
    h2 = _seg_norm_16(h1, dis, e2(edge_index_gg[0]), e2(edge_index_gg[1]), _Z16)
    out = (x_game @ p['tag_W'][0] + h1[:, :5] @ p['tag_W'][1]
           + h2[:, :5] @ p['tag_W'][2])
    g = jax.nn.relu(out + p['tag_b'])

    s_sum, cnt8 = _seg_plain_16(_pad16(x_state), e2(edge_index_ss[0]),
                                e2(edge_index_ss[1]), _Z16, _Z8, _ONES8)
    mean = s_sum[:, :6] / jnp.maximum(fold(cnt8), 1.0)[:, None]
    s = jax.nn.relu(mean @ p['s2_Wl'] + x_state @ p['s2_Wr'] + p['s2_b'])

    hs = g @ p['g_Ws']
    asn = hs @ p['g_as']
    adn = s @ (p['g_Wd'] @ p['g_ad'])
    aen = edge_attr @ (p['g_We'] @ p['g_ae'])
    bnd = jnp.max(asn) + jnp.max(adn) + jnp.max(aen)
    bnd = jnp.maximum(bnd, 0.2 * bnd)
    bnd16 = jnp.full((16,), bnd)
    u = _seg_gat(hs, asn, adn, e2(aen), bnd16,
                 e2(edge_index_hist[0]), e2(edge_index_hist[1]), _Z64)
    den = _seg_wcount(asn, adn, e2(aen), bnd16,
                      e2(edge_index_hist[0]), e2(edge_index_hist[1]), _Z16)
    s = jax.nn.relu(u / jnp.maximum(fold(den), 1e-16)[:, None] + p['g_b'])

    in_sum, icnt8 = _seg_plain_64(g, e2(edge_index_in[0]),
                                  e2(edge_index_in[1]), _Z64, _Z8, _ONES8)
    mean = in_sum / jnp.maximum(fold(icnt8), 1.0)[:, None]
    s = jax.nn.relu(mean @ p['s4_Wl'] + s @ p['s4_Wr'] + p['s4_b'])

    s = s @ p['lin_W'] + p['lin_b']
    logit = (s @ p['ll_W'] + p['ll_b'])[:, 0]
    state_out = _softmax_over_all(logit)[:, None]
    return (state_out, x_game)
